# Initial kernel scaffold; baseline (speedup 1.0000x reference)
import jax
import jax.numpy as jnp
from jax.experimental import pallas as pl


def kernel(input_tensor, segment_tensor, tok_table, seg_table, pos_table, gamma, beta):
    raise NotImplementedError("write your pallas kernel here")


# speedup vs baseline: 6.2783x; 6.2783x over previous
import functools

import jax
import jax.numpy as jnp
from jax import lax
from jax.experimental import pallas as pl
from jax.experimental.pallas import tpu as pltpu
from jax.experimental.pallas import tpu_sc as plsc

_NC, _NS = 2, 16
_NW = _NC * _NS
_LANE = 128


def _sc_gather(idx3, table):
    nw, ch, lane = idx3.shape
    d = table.shape[1]
    mesh = plsc.VectorSubcoreMesh(core_axis_name="c", subcore_axis_name="s")

    @functools.partial(
        pl.kernel,
        out_type=jax.ShapeDtypeStruct((nw * ch, lane, d), jnp.float32),
        mesh=mesh,
        compiler_params=pltpu.CompilerParams(use_tc_tiling_on_sc=False),
        scratch_types=[
            pltpu.VMEM((ch, lane), jnp.int32),
            pltpu.VMEM((lane, d), jnp.float32),
            pltpu.SemaphoreType.DMA,
        ],
    )
    def k(idx_hbm, table_hbm, out_hbm, idx_v, buf, sem):
        w = lax.axis_index("s") * _NC + lax.axis_index("c")
        pltpu.sync_copy(idx_hbm.at[w], idx_v)

        def body(j, carry):
            pltpu.async_copy(table_hbm.at[idx_v.at[j]], buf, sem).wait()
            pltpu.sync_copy(buf, out_hbm.at[w * ch + j])
            return carry

        lax.fori_loop(0, ch, body, 0)

    return k(idx3, table)


def _tc_add_ln(gathered, segment, pos_sub, seg01, gamma2, beta2):
    b, s, d = gathered.shape
    bb = 8

    def body(g_ref, seg_ref, pos_ref, s01_ref, gam_ref, bet_ref, o_ref):
        x = g_ref[...]
        seg = seg_ref[...]
        s0 = s01_ref[0:1, :]
        s1 = s01_ref[1:2, :]
        x = x + pos_ref[...][None, :, :]
        x = x + jnp.where(seg[:, :, None] == 0, s0[None, :, :], s1[None, :, :])
        mean = jnp.mean(x, axis=-1, keepdims=True)
        xc = x - mean
        var = jnp.mean(xc * xc, axis=-1, keepdims=True)
        y = xc * lax.rsqrt(var + 1e-5)
        o_ref[...] = y * gam_ref[...][None, :, :] + bet_ref[...][None, :, :]

    return pl.pallas_call(
        body,
        grid=(b // bb,),
        in_specs=[
            pl.BlockSpec((bb, s, d), lambda i: (i, 0, 0)),
            pl.BlockSpec((bb, s), lambda i: (i, 0)),
            pl.BlockSpec((s, d), lambda i: (0, 0)),
            pl.BlockSpec((2, d), lambda i: (0, 0)),
            pl.BlockSpec((1, d), lambda i: (0, 0)),
            pl.BlockSpec((1, d), lambda i: (0, 0)),
        ],
        out_specs=pl.BlockSpec((bb, s, d), lambda i: (i, 0, 0)),
        out_shape=jax.ShapeDtypeStruct((b, s, d), jnp.float32),
    )(gathered, segment, pos_sub, seg01, gamma2, beta2)


def _tc_add_ln_pairs(gathered2, segf2, base_tile, dseg2, gam2, bet2,
                     sel, selt, rblk):
    n2 = gathered2.shape[0]
    d = 64

    def body(g_ref, seg_ref, base_ref, dseg_ref, gam_ref, bet_ref,
             sel_ref, selt_ref, o_ref):
        x = g_ref[...]
        t2 = seg_ref[...]
        sel_m = sel_ref[...]
        selt_m = selt_ref[0:2, :]
        tb = jax.lax.dot(t2, selt_m)
        x = x + base_ref[...] + tb * dseg_ref[...]
        s1 = jax.lax.dot(x, sel_m)
        s2 = jax.lax.dot(x * x, sel_m)
        mean = s1 * (1.0 / d)
        var = s2 * (1.0 / d) - mean * mean
        rs = jax.lax.rsqrt(var + 1e-5)
        rsb = jax.lax.dot(rs, selt_m)
        cb = jax.lax.dot(mean * rs, selt_m)
        o_ref[...] = (x * rsb - cb) * gam_ref[...] + bet_ref[...]

    return pl.pallas_call(
        body,
        grid=(n2 // rblk,),
        in_specs=[
            pl.BlockSpec((rblk, 128), lambda i: (i, 0)),
            pl.BlockSpec((rblk, 2), lambda i: (i, 0)),
            pl.BlockSpec((rblk, 128), lambda i: (0, 0)),
            pl.BlockSpec((1, 128), lambda i: (0, 0)),
            pl.BlockSpec((1, 128), lambda i: (0, 0)),
            pl.BlockSpec((1, 128), lambda i: (0, 0)),
            pl.BlockSpec((128, 2), lambda i: (0, 0)),
            pl.BlockSpec((8, 128), lambda i: (0, 0)),
        ],
        out_specs=pl.BlockSpec((rblk, 128), lambda i: (i, 0)),
        out_shape=jax.ShapeDtypeStruct((n2, 128), jnp.float32),
    )(gathered2, segf2, base_tile, dseg2, gam2, bet2, sel, selt)


def kernel(input_tensor, segment_tensor, tok_table, seg_table, pos_table,
           gamma, beta):
    b, s = input_tensor.shape
    d = tok_table.shape[1]
    n = b * s
    ch = n // (_NW * _LANE)
    idx3 = input_tensor.reshape(_NW, ch, _LANE)
    gathered = _sc_gather(idx3, tok_table)

    n2 = n // 2
    g2 = gathered.reshape(n2, 128)
    segf2 = segment_tensor.astype(jnp.float32).reshape(n2, 2)
    period = s // 2
    rblk = 1600
    base = pos_table[:s].reshape(period, 2 * d) + jnp.tile(seg_table[0], 2)[None, :]
    base_tile = jnp.tile(base, (rblk // period, 1))
    dseg2 = jnp.tile(seg_table[1] - seg_table[0], 2).reshape(1, 2 * d)
    gam2 = jnp.tile(gamma, 2).reshape(1, 2 * d)
    bet2 = jnp.tile(beta, 2).reshape(1, 2 * d)
    half = jnp.arange(2 * d, dtype=jnp.int32) // d
    sel = (half[:, None] == jnp.arange(2)[None, :]).astype(jnp.float32)
    selt = jnp.zeros((8, 2 * d), jnp.float32).at[0:2, :].set(sel.T)
    out2 = _tc_add_ln_pairs(g2, segf2, base_tile, dseg2, gam2, bet2,
                            sel, selt, rblk)
    return out2.reshape(b, s, d)


# speedup vs baseline: 6.7036x; 1.0677x over previous
import functools

import jax
import jax.numpy as jnp
from jax import lax
from jax.experimental import pallas as pl
from jax.experimental.pallas import tpu as pltpu
from jax.experimental.pallas import tpu_sc as plsc

_NC, _NS = 2, 16
_NW = _NC * _NS
_LANE = 128


def _sc_gather(idx2, table, ch):
    nrow, lane = idx2.shape
    nw = nrow // ch
    d = table.shape[1]
    mesh = plsc.VectorSubcoreMesh(core_axis_name="c", subcore_axis_name="s")

    @functools.partial(
        pl.kernel,
        out_type=jax.ShapeDtypeStruct((nrow, lane, d), jnp.float32),
        mesh=mesh,
        compiler_params=pltpu.CompilerParams(use_tc_tiling_on_sc=False),
        scratch_types=[
            pltpu.VMEM((ch, lane), jnp.int32),
            pltpu.VMEM((4, lane, d), jnp.float32),
            pltpu.SemaphoreType.DMA((4,)),
            pltpu.SemaphoreType.DMA((4,)),
        ],
    )
    def k(idx_hbm, table_hbm, out_hbm, idx_v, buf, gsem, ssem):
        w = lax.axis_index("s") * _NC + lax.axis_index("c")
        pltpu.sync_copy(idx_hbm.at[pl.ds(w * ch, ch)], idx_v)

        def g_copy(j):
            b = lax.rem(j, 4)
            return pltpu.make_async_copy(
                table_hbm.at[idx_v.at[j]], buf.at[b], gsem.at[b])

        def s_copy(j):
            b = lax.rem(j, 4)
            return pltpu.make_async_copy(
                buf.at[b], out_hbm.at[w * ch + j], ssem.at[b])

        g_copy(0).start()
        g_copy(1).start()

        def body(j, carry):
            g_copy(j).wait()
            s_copy(j).start()

            @pl.when(j + 2 < ch)
            def _():
                @pl.when(j >= 2)
                def _():
                    s_copy(j - 2).wait()

                g_copy(j + 2).start()

            return carry

        lax.fori_loop(0, ch, body, 0)

        def drain(j, carry):
            s_copy(j).wait()
            return carry

        lax.fori_loop(ch - 4, ch, drain, 0)

    return k(idx2, table)


def _tc_add_ln(gathered, segment, pos_sub, seg01, gamma2, beta2):
    b, s, d = gathered.shape
    bb = 8

    def body(g_ref, seg_ref, pos_ref, s01_ref, gam_ref, bet_ref, o_ref):
        x = g_ref[...]
        seg = seg_ref[...]
        s0 = s01_ref[0:1, :]
        s1 = s01_ref[1:2, :]
        x = x + pos_ref[...][None, :, :]
        x = x + jnp.where(seg[:, :, None] == 0, s0[None, :, :], s1[None, :, :])
        mean = jnp.mean(x, axis=-1, keepdims=True)
        xc = x - mean
        var = jnp.mean(xc * xc, axis=-1, keepdims=True)
        y = xc * lax.rsqrt(var + 1e-5)
        o_ref[...] = y * gam_ref[...][None, :, :] + bet_ref[...][None, :, :]

    return pl.pallas_call(
        body,
        grid=(b // bb,),
        in_specs=[
            pl.BlockSpec((bb, s, d), lambda i: (i, 0, 0)),
            pl.BlockSpec((bb, s), lambda i: (i, 0)),
            pl.BlockSpec((s, d), lambda i: (0, 0)),
            pl.BlockSpec((2, d), lambda i: (0, 0)),
            pl.BlockSpec((1, d), lambda i: (0, 0)),
            pl.BlockSpec((1, d), lambda i: (0, 0)),
        ],
        out_specs=pl.BlockSpec((bb, s, d), lambda i: (i, 0, 0)),
        out_shape=jax.ShapeDtypeStruct((b, s, d), jnp.float32),
    )(gathered, segment, pos_sub, seg01, gamma2, beta2)


def _tc_add_ln_pairs(gathered2, segf2, base_tile, dseg2, gam2, bet2,
                     sel, selt, rblk):
    n2 = gathered2.shape[0]
    d = 64

    def body(g_ref, seg_ref, base_ref, dseg_ref, gam_ref, bet_ref,
             sel_ref, selt_ref, o_ref):
        x = g_ref[...]
        t2 = seg_ref[...]
        sel_m = sel_ref[...]
        selt_m = selt_ref[0:2, :]
        tb = jax.lax.dot(t2, selt_m)
        x = x + base_ref[...] + tb * dseg_ref[...]
        s1 = jax.lax.dot(x, sel_m)
        s2 = jax.lax.dot(x * x, sel_m)
        mean = s1 * (1.0 / d)
        var = s2 * (1.0 / d) - mean * mean
        rs = jax.lax.rsqrt(var + 1e-5)
        rsb = jax.lax.dot(rs, selt_m)
        cb = jax.lax.dot(mean * rs, selt_m)
        o_ref[...] = (x * rsb - cb) * gam_ref[...] + bet_ref[...]

    return pl.pallas_call(
        body,
        grid=(n2 // rblk,),
        in_specs=[
            pl.BlockSpec((rblk, 128), lambda i: (i, 0)),
            pl.BlockSpec((rblk, 2), lambda i: (i, 0)),
            pl.BlockSpec((rblk, 128), lambda i: (0, 0)),
            pl.BlockSpec((1, 128), lambda i: (0, 0)),
            pl.BlockSpec((1, 128), lambda i: (0, 0)),
            pl.BlockSpec((1, 128), lambda i: (0, 0)),
            pl.BlockSpec((128, 2), lambda i: (0, 0)),
            pl.BlockSpec((8, 128), lambda i: (0, 0)),
        ],
        out_specs=pl.BlockSpec((rblk, 128), lambda i: (i, 0)),
        out_shape=jax.ShapeDtypeStruct((n2, 128), jnp.float32),
    )(gathered2, segf2, base_tile, dseg2, gam2, bet2, sel, selt)


def kernel(input_tensor, segment_tensor, tok_table, seg_table, pos_table,
           gamma, beta):
    b, s = input_tensor.shape
    d = tok_table.shape[1]
    n = b * s
    ch = n // (_NW * _LANE)
    idx2 = input_tensor.reshape(_NW * ch, _LANE)
    gathered = _sc_gather(idx2, tok_table, ch)

    n2 = n // 2
    g2 = gathered.reshape(n2, 128)
    segf2 = segment_tensor.astype(jnp.float32).reshape(n2, 2)
    period = s // 2
    rblk = 1600
    base = pos_table[:s].reshape(period, 2 * d) + jnp.tile(seg_table[0], 2)[None, :]
    base_tile = jnp.tile(base, (rblk // period, 1))
    dseg2 = jnp.tile(seg_table[1] - seg_table[0], 2).reshape(1, 2 * d)
    gam2 = jnp.tile(gamma, 2).reshape(1, 2 * d)
    bet2 = jnp.tile(beta, 2).reshape(1, 2 * d)
    half = jnp.arange(2 * d, dtype=jnp.int32) // d
    sel = (half[:, None] == jnp.arange(2)[None, :]).astype(jnp.float32)
    selt = jnp.zeros((8, 2 * d), jnp.float32).at[0:2, :].set(sel.T)
    out2 = _tc_add_ln_pairs(g2, segf2, base_tile, dseg2, gam2, bet2,
                            sel, selt, rblk)
    return out2.reshape(b, s, d)


# speedup vs baseline: 6.8742x; 1.0254x over previous
import functools

import jax
import jax.numpy as jnp
from jax import lax
from jax.experimental import pallas as pl
from jax.experimental.pallas import tpu as pltpu
from jax.experimental.pallas import tpu_sc as plsc

_NC, _NS = 2, 16
_NW = _NC * _NS
_LANE = 128


def _sc_gather(idx2, table, ch):
    nrow, lane = idx2.shape
    nw = nrow // ch
    d = table.shape[1]
    mesh = plsc.VectorSubcoreMesh(core_axis_name="c", subcore_axis_name="s")

    @functools.partial(
        pl.kernel,
        out_type=jax.ShapeDtypeStruct((nrow, lane, d), jnp.float32),
        mesh=mesh,
        compiler_params=pltpu.CompilerParams(use_tc_tiling_on_sc=False),
        scratch_types=[
            pltpu.VMEM((ch, lane), jnp.int32),
            pltpu.VMEM((4, lane, d), jnp.float32),
            pltpu.SemaphoreType.DMA((4,)),
            pltpu.SemaphoreType.DMA((4,)),
        ],
    )
    def k(idx_hbm, table_hbm, out_hbm, idx_v, buf, gsem, ssem):
        w = lax.axis_index("s") * _NC + lax.axis_index("c")
        pltpu.sync_copy(idx_hbm.at[pl.ds(w * ch, ch)], idx_v)

        def g_copy(j):
            b = lax.rem(j, 4)
            return pltpu.make_async_copy(
                table_hbm.at[idx_v.at[j]], buf.at[b], gsem.at[b])

        def s_copy(j):
            b = lax.rem(j, 4)
            return pltpu.make_async_copy(
                buf.at[b], out_hbm.at[w * ch + j], ssem.at[b])

        g_copy(0).start()
        g_copy(1).start()

        def body(j, carry):
            g_copy(j).wait()
            s_copy(j).start()

            @pl.when(j + 2 < ch)
            def _():
                @pl.when(j >= 2)
                def _():
                    s_copy(j - 2).wait()

                g_copy(j + 2).start()

            return carry

        lax.fori_loop(0, ch, body, 0)

        def drain(j, carry):
            s_copy(j).wait()
            return carry

        lax.fori_loop(ch - 4, ch, drain, 0)

    return k(idx2, table)


def _tc_add_ln(gathered, segment, pos_sub, seg01, gamma2, beta2):
    b, s, d = gathered.shape
    bb = 8

    def body(g_ref, seg_ref, pos_ref, s01_ref, gam_ref, bet_ref, o_ref):
        x = g_ref[...]
        seg = seg_ref[...]
        s0 = s01_ref[0:1, :]
        s1 = s01_ref[1:2, :]
        x = x + pos_ref[...][None, :, :]
        x = x + jnp.where(seg[:, :, None] == 0, s0[None, :, :], s1[None, :, :])
        mean = jnp.mean(x, axis=-1, keepdims=True)
        xc = x - mean
        var = jnp.mean(xc * xc, axis=-1, keepdims=True)
        y = xc * lax.rsqrt(var + 1e-5)
        o_ref[...] = y * gam_ref[...][None, :, :] + bet_ref[...][None, :, :]

    return pl.pallas_call(
        body,
        grid=(b // bb,),
        in_specs=[
            pl.BlockSpec((bb, s, d), lambda i: (i, 0, 0)),
            pl.BlockSpec((bb, s), lambda i: (i, 0)),
            pl.BlockSpec((s, d), lambda i: (0, 0)),
            pl.BlockSpec((2, d), lambda i: (0, 0)),
            pl.BlockSpec((1, d), lambda i: (0, 0)),
            pl.BlockSpec((1, d), lambda i: (0, 0)),
        ],
        out_specs=pl.BlockSpec((bb, s, d), lambda i: (i, 0, 0)),
        out_shape=jax.ShapeDtypeStruct((b, s, d), jnp.float32),
    )(gathered, segment, pos_sub, seg01, gamma2, beta2)


def _tc_add_ln_pairs(gathered2, segf2, base_tile, dseg2, gam2, bet2,
                     sel, selt, lmat, mmat, emat, rblk):
    n2 = gathered2.shape[0]
    d = 64

    def body(g_ref, seg_ref, base_ref, dseg_ref, gam_ref, bet_ref,
             sel_ref, selt_ref, lmat_ref, mmat_ref, emat_ref, o_ref):
        x = g_ref[...]
        segi = seg_ref[...]
        segf = segi.astype(jnp.float32)
        sel_m = sel_ref[...]
        selt_m = selt_ref[0:2, :]
        t_rows = jax.lax.dot(lmat_ref[...], segf)
        t2 = jax.lax.dot(t_rows * mmat_ref[...], emat_ref[...])
        tb = jax.lax.dot(t2, selt_m)
        x = x + base_ref[...] + tb * dseg_ref[...]
        s1 = jax.lax.dot(x, sel_m)
        s2 = jax.lax.dot(x * x, sel_m)
        mean = s1 * (1.0 / d)
        var = s2 * (1.0 / d) - mean * mean
        rs = jax.lax.rsqrt(var + 1e-5)
        rsb = jax.lax.dot(rs, selt_m)
        cb = jax.lax.dot(mean * rs, selt_m)
        o_ref[...] = (x * rsb - cb) * gam_ref[...] + bet_ref[...]

    s = segf2.shape[1]
    brows = 2 * rblk // s
    return pl.pallas_call(
        body,
        grid=(n2 // rblk,),
        in_specs=[
            pl.BlockSpec((rblk, 128), lambda i: (i, 0)),
            pl.BlockSpec((brows, s), lambda i: (i, 0)),
            pl.BlockSpec((rblk, 128), lambda i: (0, 0)),
            pl.BlockSpec((1, 128), lambda i: (0, 0)),
            pl.BlockSpec((1, 128), lambda i: (0, 0)),
            pl.BlockSpec((1, 128), lambda i: (0, 0)),
            pl.BlockSpec((128, 2), lambda i: (0, 0)),
            pl.BlockSpec((8, 128), lambda i: (0, 0)),
            pl.BlockSpec((rblk, brows), lambda i: (0, 0)),
            pl.BlockSpec((rblk, s), lambda i: (0, 0)),
            pl.BlockSpec((s, 2), lambda i: (0, 0)),
        ],
        out_specs=pl.BlockSpec((rblk, 128), lambda i: (i, 0)),
        out_shape=jax.ShapeDtypeStruct((n2, 128), jnp.float32),
    )(gathered2, segf2, base_tile, dseg2, gam2, bet2, sel, selt,
      lmat, mmat, emat)


def kernel(input_tensor, segment_tensor, tok_table, seg_table, pos_table,
           gamma, beta):
    b, s = input_tensor.shape
    d = tok_table.shape[1]
    n = b * s
    ch = n // (_NW * _LANE)
    idx2 = input_tensor.reshape(_NW * ch, _LANE)
    gathered = _sc_gather(idx2, tok_table, ch)

    n2 = n // 2
    g2 = gathered.reshape(n2, 128)
    segf2 = segment_tensor
    period = s // 2
    rblk = 1600
    base = pos_table[:s].reshape(period, 2 * d) + jnp.tile(seg_table[0], 2)[None, :]
    base_tile = jnp.tile(base, (rblk // period, 1))
    dseg2 = jnp.tile(seg_table[1] - seg_table[0], 2).reshape(1, 2 * d)
    gam2 = jnp.tile(gamma, 2).reshape(1, 2 * d)
    bet2 = jnp.tile(beta, 2).reshape(1, 2 * d)
    half = jnp.arange(2 * d, dtype=jnp.int32) // d
    sel = (half[:, None] == jnp.arange(2)[None, :]).astype(jnp.float32)
    selt = jnp.zeros((8, 2 * d), jnp.float32).at[0:2, :].set(sel.T)
    brows = 2 * rblk // s
    r_ids = jnp.arange(rblk, dtype=jnp.int32)
    lmat = (r_ids[:, None] // period
            == jnp.arange(brows, dtype=jnp.int32)[None, :]).astype(jnp.float32)
    mmat = (jnp.arange(s, dtype=jnp.int32)[None, :] // 2
            == (r_ids % period)[:, None]).astype(jnp.float32)
    emat = (jnp.arange(s, dtype=jnp.int32)[:, None] % 2
            == jnp.arange(2, dtype=jnp.int32)[None, :]).astype(jnp.float32)
    out2 = _tc_add_ln_pairs(g2, segf2, base_tile, dseg2, gam2, bet2,
                            sel, selt, lmat, mmat, emat, rblk)
    return out2.reshape(b, s, d)
